# Initial kernel scaffold; baseline (speedup 1.0000x reference)
import jax
import jax.numpy as jnp
from jax.experimental import pallas as pl


def kernel(cls_logits_0, reg_preds_0, ctr_logits_0, cls_logits_1, reg_preds_1, ctr_logits_1, cls_logits_2, reg_preds_2, ctr_logits_2, cls_logits_3, reg_preds_3, ctr_logits_3, cls_logits_4, reg_preds_4, ctr_logits_4, imgs):
    raise NotImplementedError("write your pallas kernel here")


# speedup vs baseline: 2.7038x; 2.7038x over previous
import jax
import jax.numpy as jnp
from jax import lax
from jax.experimental import pallas as pl
from jax.experimental.pallas import tpu as pltpu

_STRIDES = (8, 16, 32, 64, 128)
_IMG = 1024
_MAX_BOXES = 1000
_SCORE_THR = 0.05
_IOU_THR = 0.6
_CHUNK = 1024
_NLOC = sum((_IMG // s) ** 2 for s in _STRIDES)
_NCHUNK = -(-_NLOC // _CHUNK)
_NPAD = _NCHUNK * _CHUNK


def _sigmoid(x):
    return jax.nn.sigmoid(x)


def _score_decode_body(stride, cls_ref, reg_ref, ctr_ref,
                       score_ref, label_ref, x1_ref, y1_ref, x2_ref, y2_ref):
    cls = cls_ref[0]
    c_dim, h, w = cls.shape
    m = jnp.max(cls, axis=0)
    iota_c = lax.broadcasted_iota(jnp.int32, cls.shape, 0)
    big = jnp.int32(1 << 30)
    lbl = jnp.min(jnp.where(cls == m[None], iota_c, big), axis=0) + 1
    ctr = ctr_ref[0, 0]
    score = jnp.sqrt(_sigmoid(m) * _sigmoid(ctr))
    f = reg_ref[0]
    xs = lax.broadcasted_iota(jnp.int32, (h, w), 1).astype(jnp.float32)
    ys = lax.broadcasted_iota(jnp.int32, (h, w), 0).astype(jnp.float32)
    s = jnp.float32(stride)
    cx = (xs + 0.5) * s
    cy = (ys + 0.5) * s
    score_ref[0] = score
    label_ref[0] = lbl
    x1_ref[0] = cx - f[0] * s
    y1_ref[0] = cy - f[1] * s
    x2_ref[0] = cx + f[2] * s
    y2_ref[0] = cy + f[3] * s


def _score_decode(cls, reg, ctr, stride, interpret=False):
    b, c_dim, h, w = cls.shape
    blk = lambda ch: pl.BlockSpec((1, ch, h, w), lambda i: (i, 0, 0, 0))
    out2 = pl.BlockSpec((1, h, w), lambda i: (i, 0, 0))
    f32 = jnp.float32
    outs = pl.pallas_call(
        lambda *a: _score_decode_body(stride, *a),
        grid=(b,),
        in_specs=[blk(c_dim), blk(4), blk(1)],
        out_specs=[out2] * 6,
        out_shape=[jax.ShapeDtypeStruct((b, h, w), f32),
                   jax.ShapeDtypeStruct((b, h, w), jnp.int32),
                   jax.ShapeDtypeStruct((b, h, w), f32),
                   jax.ShapeDtypeStruct((b, h, w), f32),
                   jax.ShapeDtypeStruct((b, h, w), f32),
                   jax.ShapeDtypeStruct((b, h, w), f32)],
        interpret=interpret,
    )(cls, reg, ctr)
    return outs


def _before(ka, ia, kb, ib):
    return (ka > kb) | ((ka == kb) & (ia < ib))


def _cmpx(arrs, j, k_bit, asc, rowi, lanei, flat):
    if j >= 128:
        d = j // 128
        ax, size = 0, 8
        low = (rowi & d) == 0
    else:
        d = j
        ax, size = 1, 128
        low = (lanei & d) == 0
    partner = tuple(
        jnp.where(low,
                  pltpu.roll(a, (size - d) % size, ax),
                  pltpu.roll(a, d, ax))
        for a in arrs)
    desc = (flat & k_bit) == 0
    if asc:
        desc = ~desc
    bef = _before(arrs[0], arrs[1], partner[0], partner[1])
    keep_a = bef ^ (desc ^ low)
    return tuple(jnp.where(keep_a, a, b) for a, b in zip(arrs, partner))


def _sort_chunk_asc(arrs, rowi, lanei, flat):
    k = 2
    while k <= _CHUNK:
        j = k // 2
        while j >= 1:
            arrs = _cmpx(arrs, j, k, True, rowi, lanei, flat)
            j //= 2
        k *= 2
    return arrs


def _merge_desc(arrs, rowi, lanei, flat):
    j = _CHUNK // 2
    while j >= 1:
        arrs = _cmpx(arrs, j, 2 * _CHUNK, False, rowi, lanei, flat)
        j //= 2
    return arrs


def _topk_body(score_ref, label_ref, x1_ref, y1_ref, x2_ref, y2_ref,
               os_ref, ol_ref, ox1_ref, oy1_ref, ox2_ref, oy2_ref):
    rowi = lax.broadcasted_iota(jnp.int32, (8, 128), 0)
    lanei = lax.broadcasted_iota(jnp.int32, (8, 128), 1)
    flat = rowi * 128 + lanei

    def step(c, buf):
        arrs = (score_ref[0, c],
                flat + c * _CHUNK,
                label_ref[0, c],
                x1_ref[0, c], y1_ref[0, c], x2_ref[0, c], y2_ref[0, c])
        arrs = _sort_chunk_asc(arrs, rowi, lanei, flat)
        bef = _before(buf[0], buf[1], arrs[0], arrs[1])
        merged = tuple(jnp.where(bef, a, b) for a, b in zip(buf, arrs))
        return _merge_desc(merged, rowi, lanei, flat)

    zf = jnp.zeros((8, 128), jnp.float32)
    buf0 = (jnp.full((8, 128), -2.0, jnp.float32),
            flat + jnp.int32(1 << 26),
            jnp.zeros((8, 128), jnp.int32),
            zf, zf, zf, zf)
    buf = lax.fori_loop(0, _NCHUNK, step, buf0)
    os_ref[0] = buf[0]
    ol_ref[0] = buf[2]
    ox1_ref[0] = buf[3]
    oy1_ref[0] = buf[4]
    ox2_ref[0] = buf[5]
    oy2_ref[0] = buf[6]


def _topk(score, label, x1, y1, x2, y2, interpret=False):
    b = score.shape[0]
    inb = pl.BlockSpec((1, _NCHUNK, 8, 128), lambda i: (i, 0, 0, 0))
    outb = pl.BlockSpec((1, 8, 128), lambda i: (i, 0, 0))
    f32 = jnp.float32
    sh = lambda dt: jax.ShapeDtypeStruct((b, 8, 128), dt)
    return pl.pallas_call(
        _topk_body,
        grid=(b,),
        in_specs=[inb] * 6,
        out_specs=[outb] * 6,
        out_shape=[sh(f32), sh(jnp.int32), sh(f32), sh(f32), sh(f32), sh(f32)],
        interpret=interpret,
    )(score, label, x1, y1, x2, y2)


def _nms_body(score_ref, label_ref, x1_ref, y1_ref, x2_ref, y2_ref,
              keep_ref, os_ref, ol_ref, ox1_ref, oy1_ref, ox2_ref, oy2_ref):
    s = score_ref[0]
    lbl = label_ref[0]
    x1 = x1_ref[0]
    y1 = y1_ref[0]
    x2 = x2_ref[0]
    y2 = y2_ref[0]
    rowi = lax.broadcasted_iota(jnp.int32, (8, 128), 0)
    lanei = lax.broadcasted_iota(jnp.int32, (8, 128), 1)
    flat = rowi * 128 + lanei
    active = flat < _MAX_BOXES
    valid = (s >= _SCORE_THR) & active

    neg = jnp.float32(-1e9)
    pos = jnp.float32(1e9)
    mx = jnp.max(jnp.maximum(jnp.maximum(jnp.where(valid, x1, neg),
                                         jnp.where(valid, y1, neg)),
                             jnp.maximum(jnp.where(valid, x2, neg),
                                         jnp.where(valid, y2, neg))))
    mn = jnp.min(jnp.minimum(jnp.minimum(jnp.where(valid, x1, pos),
                                         jnp.where(valid, y1, pos)),
                             jnp.minimum(jnp.where(valid, x2, pos),
                                         jnp.where(valid, y2, pos))))
    off = lbl.astype(jnp.float32) * (mx - mn + 1.0)
    sx1 = x1 + off
    sy1 = y1 + off
    sx2 = x2 + off
    sy2 = y2 + off
    area = jnp.maximum(sx2 - sx1, 0.0) * jnp.maximum(sy2 - sy1, 0.0)

    def body(i, keep):
        oh = flat == i
        ohf = jnp.where(oh, 1.0, 0.0)
        bx1 = jnp.sum(sx1 * ohf)
        by1 = jnp.sum(sy1 * ohf)
        bx2 = jnp.sum(sx2 * ohf)
        by2 = jnp.sum(sy2 * ohf)
        ba = jnp.sum(area * ohf)
        ki = jnp.sum(keep * ohf) > 0.0
        ltx = jnp.maximum(bx1, sx1)
        lty = jnp.maximum(by1, sy1)
        rbx = jnp.minimum(bx2, sx2)
        rby = jnp.minimum(by2, sy2)
        inter = jnp.maximum(rbx - ltx, 0.0) * jnp.maximum(rby - lty, 0.0)
        iou = inter / (ba + area - inter + 1e-9)
        sup = (iou > _IOU_THR) & (flat > i) & ki
        return jnp.where(sup, 0.0, keep)

    keep0 = jnp.where(valid, 1.0, 0.0)
    keepf = lax.fori_loop(0, _MAX_BOXES, body, keep0)
    keepi = keepf.astype(jnp.int32)
    lim = jnp.float32(_IMG - 1)
    keep_ref[0] = keepi
    os_ref[0] = s * keepf
    ol_ref[0] = lbl * keepi
    ox1_ref[0] = jnp.clip(x1, 0.0, lim) * keepf
    oy1_ref[0] = jnp.clip(y1, 0.0, lim) * keepf
    ox2_ref[0] = jnp.clip(x2, 0.0, lim) * keepf
    oy2_ref[0] = jnp.clip(y2, 0.0, lim) * keepf


def _nms(score, label, x1, y1, x2, y2, interpret=False):
    b = score.shape[0]
    bspec = pl.BlockSpec((1, 8, 128), lambda i: (i, 0, 0))
    f32 = jnp.float32
    sh = lambda dt: jax.ShapeDtypeStruct((b, 8, 128), dt)
    return pl.pallas_call(
        _nms_body,
        grid=(b,),
        in_specs=[bspec] * 6,
        out_specs=[bspec] * 7,
        out_shape=[sh(jnp.int32), sh(f32), sh(jnp.int32),
                   sh(f32), sh(f32), sh(f32), sh(f32)],
        interpret=interpret,
    )(score, label, x1, y1, x2, y2)


def _run(cls_list, reg_list, ctr_list, interpret=False):
    b = cls_list[0].shape[0]
    per_level = [_score_decode(c, r, t, s, interpret)
                 for c, r, t, s in zip(cls_list, reg_list, ctr_list, _STRIDES)]

    def flat_cat(k, pad_val, dtype):
        parts = [lev[k].reshape(b, -1) for lev in per_level]
        full = jnp.concatenate(parts, axis=1)
        full = jnp.pad(full, ((0, 0), (0, _NPAD - _NLOC)),
                       constant_values=pad_val).astype(dtype)
        return full.reshape(b, _NCHUNK, 8, 128)

    score = flat_cat(0, -1.0, jnp.float32)
    label = flat_cat(1, 0, jnp.int32)
    x1 = flat_cat(2, 0.0, jnp.float32)
    y1 = flat_cat(3, 0.0, jnp.float32)
    x2 = flat_cat(4, 0.0, jnp.float32)
    y2 = flat_cat(5, 0.0, jnp.float32)

    ts, tl, tx1, ty1, tx2, ty2 = _topk(score, label, x1, y1, x2, y2, interpret)
    keep, os_, ol, ox1, oy1, ox2, oy2 = _nms(ts, tl, tx1, ty1, tx2, ty2,
                                             interpret)

    cut = lambda a: a.reshape(b, _CHUNK)[:, :_MAX_BOXES]
    out_scores = cut(os_)
    out_labels = cut(ol)
    out_boxes = jnp.stack([cut(ox1), cut(oy1), cut(ox2), cut(oy2)], axis=-1)
    keep_b = cut(keep).astype(bool)
    return out_scores, out_labels, out_boxes, keep_b


def kernel(cls_logits_0, reg_preds_0, ctr_logits_0,
           cls_logits_1, reg_preds_1, ctr_logits_1,
           cls_logits_2, reg_preds_2, ctr_logits_2,
           cls_logits_3, reg_preds_3, ctr_logits_3,
           cls_logits_4, reg_preds_4, ctr_logits_4,
           imgs):
    cls_list = [cls_logits_0, cls_logits_1, cls_logits_2,
                cls_logits_3, cls_logits_4]
    reg_list = [reg_preds_0, reg_preds_1, reg_preds_2,
                reg_preds_3, reg_preds_4]
    ctr_list = [ctr_logits_0, ctr_logits_1, ctr_logits_2,
                ctr_logits_3, ctr_logits_4]
    return _run(cls_list, reg_list, ctr_list)


# speedup vs baseline: 2.8676x; 1.0606x over previous
import jax
import jax.numpy as jnp
from jax import lax
from jax.experimental import pallas as pl
from jax.experimental.pallas import tpu as pltpu

_STRIDES = (8, 16, 32, 64, 128)
_IMG = 1024
_MAX_BOXES = 1000
_SCORE_THR = 0.05
_IOU_THR = 0.6
_CHUNK = 1024
_NLOC = sum((_IMG // s) ** 2 for s in _STRIDES)
_NCHUNK = -(-_NLOC // _CHUNK)
_NPAD = _NCHUNK * _CHUNK


def _sigmoid(x):
    return jax.nn.sigmoid(x)


def _score_decode_body(stride, cls_ref, reg_ref, ctr_ref,
                       score_ref, label_ref, x1_ref, y1_ref, x2_ref, y2_ref):
    cls = cls_ref[0]
    c_dim, h, w = cls.shape
    m = jnp.max(cls, axis=0)
    iota_c = lax.broadcasted_iota(jnp.int32, cls.shape, 0)
    big = jnp.int32(1 << 30)
    lbl = jnp.min(jnp.where(cls == m[None], iota_c, big), axis=0) + 1
    ctr = ctr_ref[0, 0]
    score = jnp.sqrt(_sigmoid(m) * _sigmoid(ctr))
    f = reg_ref[0]
    xs = lax.broadcasted_iota(jnp.int32, (h, w), 1).astype(jnp.float32)
    ys = lax.broadcasted_iota(jnp.int32, (h, w), 0).astype(jnp.float32)
    s = jnp.float32(stride)
    cx = (xs + 0.5) * s
    cy = (ys + 0.5) * s
    score_ref[0] = score
    label_ref[0] = lbl
    x1_ref[0] = cx - f[0] * s
    y1_ref[0] = cy - f[1] * s
    x2_ref[0] = cx + f[2] * s
    y2_ref[0] = cy + f[3] * s


def _score_decode(cls, reg, ctr, stride, interpret=False):
    b, c_dim, h, w = cls.shape
    blk = lambda ch: pl.BlockSpec((1, ch, h, w), lambda i: (i, 0, 0, 0))
    out2 = pl.BlockSpec((1, h, w), lambda i: (i, 0, 0))
    f32 = jnp.float32
    outs = pl.pallas_call(
        lambda *a: _score_decode_body(stride, *a),
        grid=(b,),
        in_specs=[blk(c_dim), blk(4), blk(1)],
        out_specs=[out2] * 6,
        out_shape=[jax.ShapeDtypeStruct((b, h, w), f32),
                   jax.ShapeDtypeStruct((b, h, w), jnp.int32),
                   jax.ShapeDtypeStruct((b, h, w), f32),
                   jax.ShapeDtypeStruct((b, h, w), f32),
                   jax.ShapeDtypeStruct((b, h, w), f32),
                   jax.ShapeDtypeStruct((b, h, w), f32)],
        interpret=interpret,
    )(cls, reg, ctr)
    return outs


def _before(ka, ia, kb, ib):
    return (ka > kb) | ((ka == kb) & (ia < ib))


def _cmpx(arrs, j, k_bit, asc, rowi, lanei, flat):
    if j >= 128:
        d = j // 128
        ax, size = 0, 8
        low = (rowi & d) == 0
    else:
        d = j
        ax, size = 1, 128
        low = (lanei & d) == 0
    partner = tuple(
        jnp.where(low,
                  pltpu.roll(a, (size - d) % size, ax),
                  pltpu.roll(a, d, ax))
        for a in arrs)
    desc = (flat & k_bit) == 0
    if asc:
        desc = ~desc
    bef = _before(arrs[0], arrs[1], partner[0], partner[1])
    keep_a = bef ^ (desc ^ low)
    return tuple(jnp.where(keep_a, a, b) for a, b in zip(arrs, partner))


def _sort_chunk_asc(arrs, rowi, lanei, flat):
    k = 2
    while k <= _CHUNK:
        j = k // 2
        while j >= 1:
            arrs = _cmpx(arrs, j, k, True, rowi, lanei, flat)
            j //= 2
        k *= 2
    return arrs


def _merge_desc(arrs, rowi, lanei, flat):
    j = _CHUNK // 2
    while j >= 1:
        arrs = _cmpx(arrs, j, 2 * _CHUNK, False, rowi, lanei, flat)
        j //= 2
    return arrs


def _topk_body(score_ref, label_ref, x1_ref, y1_ref, x2_ref, y2_ref,
               os_ref, ol_ref, ox1_ref, oy1_ref, ox2_ref, oy2_ref):
    rowi = lax.broadcasted_iota(jnp.int32, (8, 128), 0)
    lanei = lax.broadcasted_iota(jnp.int32, (8, 128), 1)
    flat = rowi * 128 + lanei

    def step(c, buf):
        arrs = (score_ref[0, c],
                flat + c * _CHUNK,
                label_ref[0, c],
                x1_ref[0, c], y1_ref[0, c], x2_ref[0, c], y2_ref[0, c])
        arrs = _sort_chunk_asc(arrs, rowi, lanei, flat)
        bef = _before(buf[0], buf[1], arrs[0], arrs[1])
        merged = tuple(jnp.where(bef, a, b) for a, b in zip(buf, arrs))
        return _merge_desc(merged, rowi, lanei, flat)

    zf = jnp.zeros((8, 128), jnp.float32)
    buf0 = (jnp.full((8, 128), -2.0, jnp.float32),
            flat + jnp.int32(1 << 26),
            jnp.zeros((8, 128), jnp.int32),
            zf, zf, zf, zf)
    buf = lax.fori_loop(0, _NCHUNK, step, buf0)
    os_ref[0] = buf[0]
    ol_ref[0] = buf[2]
    ox1_ref[0] = buf[3]
    oy1_ref[0] = buf[4]
    ox2_ref[0] = buf[5]
    oy2_ref[0] = buf[6]


def _topk(score, label, x1, y1, x2, y2, interpret=False):
    b = score.shape[0]
    inb = pl.BlockSpec((1, _NCHUNK, 8, 128), lambda i: (i, 0, 0, 0))
    outb = pl.BlockSpec((1, 8, 128), lambda i: (i, 0, 0))
    f32 = jnp.float32
    sh = lambda dt: jax.ShapeDtypeStruct((b, 8, 128), dt)
    return pl.pallas_call(
        _topk_body,
        grid=(b,),
        in_specs=[inb] * 6,
        out_specs=[outb] * 6,
        out_shape=[sh(f32), sh(jnp.int32), sh(f32), sh(f32), sh(f32), sh(f32)],
        interpret=interpret,
    )(score, label, x1, y1, x2, y2)


def _nms_body(score_ref, label_ref, x1_ref, y1_ref, x2_ref, y2_ref,
              ls_ref, xs1_ref, ys1_ref, xs2_ref, ys2_ref,
              keep_ref, os_ref, ol_ref, ox1_ref, oy1_ref, ox2_ref, oy2_ref):
    s = score_ref[0]
    lbl = label_ref[0]
    x1 = x1_ref[0]
    y1 = y1_ref[0]
    x2 = x2_ref[0]
    y2 = y2_ref[0]
    rowi = lax.broadcasted_iota(jnp.int32, (8, 128), 0)
    lanei = lax.broadcasted_iota(jnp.int32, (8, 128), 1)
    flat = rowi * 128 + lanei
    active = flat < _MAX_BOXES
    valid = (s >= _SCORE_THR) & active

    neg = jnp.float32(-1e9)
    pos = jnp.float32(1e9)
    mx = jnp.max(jnp.maximum(jnp.maximum(jnp.where(valid, x1, neg),
                                         jnp.where(valid, y1, neg)),
                             jnp.maximum(jnp.where(valid, x2, neg),
                                         jnp.where(valid, y2, neg))))
    mn = jnp.min(jnp.minimum(jnp.minimum(jnp.where(valid, x1, pos),
                                         jnp.where(valid, y1, pos)),
                             jnp.minimum(jnp.where(valid, x2, pos),
                                         jnp.where(valid, y2, pos))))
    mxmn = mx - mn + 1.0
    off = lbl.astype(jnp.float32) * mxmn
    sx1 = x1 + off
    sy1 = y1 + off
    sx2 = x2 + off
    sy2 = y2 + off
    area = jnp.maximum(sx2 - sx1, 0.0) * jnp.maximum(sy2 - sy1, 0.0)

    def body(i, keep):
        ohf = jnp.where(flat == i, 1.0, 0.0)
        offi = ls_ref[0, 0, i].astype(jnp.float32) * mxmn
        bx1 = xs1_ref[0, 0, i] + offi
        by1 = ys1_ref[0, 0, i] + offi
        bx2 = xs2_ref[0, 0, i] + offi
        by2 = ys2_ref[0, 0, i] + offi
        ba = jnp.maximum(bx2 - bx1, 0.0) * jnp.maximum(by2 - by1, 0.0)
        ki = jnp.sum(keep * ohf) > 0.0
        ltx = jnp.maximum(bx1, sx1)
        lty = jnp.maximum(by1, sy1)
        rbx = jnp.minimum(bx2, sx2)
        rby = jnp.minimum(by2, sy2)
        inter = jnp.maximum(rbx - ltx, 0.0) * jnp.maximum(rby - lty, 0.0)
        iou = inter / (ba + area - inter + 1e-9)
        sup = (iou > _IOU_THR) & (flat > i) & ki
        return jnp.where(sup, 0.0, keep)

    keep0 = jnp.where(valid, 1.0, 0.0)
    keepf = lax.fori_loop(0, _MAX_BOXES, body, keep0)
    keepi = keepf.astype(jnp.int32)
    lim = jnp.float32(_IMG - 1)
    keep_ref[0] = keepi
    os_ref[0] = s * keepf
    ol_ref[0] = lbl * keepi
    ox1_ref[0] = jnp.clip(x1, 0.0, lim) * keepf
    oy1_ref[0] = jnp.clip(y1, 0.0, lim) * keepf
    ox2_ref[0] = jnp.clip(x2, 0.0, lim) * keepf
    oy2_ref[0] = jnp.clip(y2, 0.0, lim) * keepf


def _nms(score, label, x1, y1, x2, y2, interpret=False):
    b = score.shape[0]
    bspec = pl.BlockSpec((1, 8, 128), lambda i: (i, 0, 0))
    sspec = pl.BlockSpec((1, 1, _CHUNK), lambda i: (i, 0, 0),
                         memory_space=pltpu.SMEM)
    f32 = jnp.float32
    sh = lambda dt: jax.ShapeDtypeStruct((b, 8, 128), dt)
    flat2 = lambda a: a.reshape(b, 1, _CHUNK)
    return pl.pallas_call(
        _nms_body,
        grid=(b,),
        in_specs=[bspec] * 6 + [sspec] * 5,
        out_specs=[bspec] * 7,
        out_shape=[sh(jnp.int32), sh(f32), sh(jnp.int32),
                   sh(f32), sh(f32), sh(f32), sh(f32)],
        interpret=interpret,
    )(score, label, x1, y1, x2, y2,
      flat2(label), flat2(x1), flat2(y1), flat2(x2), flat2(y2))


def _run(cls_list, reg_list, ctr_list, interpret=False):
    b = cls_list[0].shape[0]
    per_level = [_score_decode(c, r, t, s, interpret)
                 for c, r, t, s in zip(cls_list, reg_list, ctr_list, _STRIDES)]

    def flat_cat(k, pad_val, dtype):
        parts = [lev[k].reshape(b, -1) for lev in per_level]
        full = jnp.concatenate(parts, axis=1)
        full = jnp.pad(full, ((0, 0), (0, _NPAD - _NLOC)),
                       constant_values=pad_val).astype(dtype)
        return full.reshape(b, _NCHUNK, 8, 128)

    score = flat_cat(0, -1.0, jnp.float32)
    label = flat_cat(1, 0, jnp.int32)
    x1 = flat_cat(2, 0.0, jnp.float32)
    y1 = flat_cat(3, 0.0, jnp.float32)
    x2 = flat_cat(4, 0.0, jnp.float32)
    y2 = flat_cat(5, 0.0, jnp.float32)

    ts, tl, tx1, ty1, tx2, ty2 = _topk(score, label, x1, y1, x2, y2, interpret)
    keep, os_, ol, ox1, oy1, ox2, oy2 = _nms(ts, tl, tx1, ty1, tx2, ty2,
                                             interpret)

    cut = lambda a: a.reshape(b, _CHUNK)[:, :_MAX_BOXES]
    out_scores = cut(os_)
    out_labels = cut(ol)
    out_boxes = jnp.stack([cut(ox1), cut(oy1), cut(ox2), cut(oy2)], axis=-1)
    keep_b = cut(keep).astype(bool)
    return out_scores, out_labels, out_boxes, keep_b


def kernel(cls_logits_0, reg_preds_0, ctr_logits_0,
           cls_logits_1, reg_preds_1, ctr_logits_1,
           cls_logits_2, reg_preds_2, ctr_logits_2,
           cls_logits_3, reg_preds_3, ctr_logits_3,
           cls_logits_4, reg_preds_4, ctr_logits_4,
           imgs):
    cls_list = [cls_logits_0, cls_logits_1, cls_logits_2,
                cls_logits_3, cls_logits_4]
    reg_list = [reg_preds_0, reg_preds_1, reg_preds_2,
                reg_preds_3, reg_preds_4]
    ctr_list = [ctr_logits_0, ctr_logits_1, ctr_logits_2,
                ctr_logits_3, ctr_logits_4]
    return _run(cls_list, reg_list, ctr_list)
